# baseline (speedup 1.0000x reference)
import functools

import numpy as np
import jax
import jax.numpy as jnp
from jax import lax
from jax.experimental import pallas as pl
from jax.experimental.pallas import tpu as pltpu
from jax.experimental.pallas import tpu_sc as plsc

D_MODEL = 128
MEMORY_LEN = 200
BATCH = 4096
NUM_COMBO = 25
SCALE = float(np.sqrt(D_MODEL))
EPS = 1e-5
ROWS = MEMORY_LEN * BATCH
WINDOW = 128
GRID = ROWS // WINDOW

TC_ROWS = 524288
SC_ROWS = ROWS - TC_ROWS
TC_BLK = 512


def _prep_body(pos_ref, pt_ref, lt_ref, g_ref, b_ref, xp_ref, xl_ref,
               t_ref, ta_ref, combo_ref, ci_ref):
    gamma = g_ref[...]
    beta = b_ref[...]
    pos2 = 2.0 * pos_ref[...]
    for c in range(NUM_COMBO):
        p, lab = c // 5, c % 5
        row = SCALE * (pt_ref[p:p + 1, :] + lt_ref[lab:lab + 1, :])
        x = pos2 + row
        mean = jnp.mean(x, axis=-1, keepdims=True)
        var = jnp.mean((x - mean) ** 2, axis=-1, keepdims=True)
        y = (x - mean) * lax.rsqrt(var + EPS) * gamma + beta
        t_ref[c] = y
        ta_ref[:, c, :] = y
    ci = 5 * xp_ref[...] + xl_ref[...]
    ci_ref[...] = ci
    iota_l = lax.broadcasted_iota(jnp.int32, (MEMORY_LEN, BATCH), 0)
    combo_ref[...] = ci * MEMORY_LEN + iota_l


def _prep(pos_table, pitch_table, label_table, ln_gamma, ln_beta, xp_t, xl_t):
    return pl.pallas_call(
        _prep_body,
        out_shape=[
            jax.ShapeDtypeStruct((NUM_COMBO, MEMORY_LEN, D_MODEL), jnp.float32),
            jax.ShapeDtypeStruct((MEMORY_LEN, NUM_COMBO, D_MODEL), jnp.float32),
            jax.ShapeDtypeStruct((MEMORY_LEN, BATCH), jnp.int32),
            jax.ShapeDtypeStruct((MEMORY_LEN, BATCH), jnp.int32),
        ],
    )(pos_table, pitch_table, label_table, ln_gamma, ln_beta, xp_t, xl_t)


def _tc_gather_body(ta_ref, ci_ref, o_ref):
    ci = ci_ref[0, 0]
    onehot = (ci[:, None] == lax.broadcasted_iota(
        jnp.int32, (TC_BLK, NUM_COMBO), 1)).astype(jnp.float32)
    o_ref[...] = jnp.dot(onehot, ta_ref[0],
                         preferred_element_type=jnp.float32)


def _tc_gather(ta, ci_blocks):
    n_blk = TC_ROWS // TC_BLK
    return pl.pallas_call(
        _tc_gather_body,
        grid=(n_blk,),
        in_specs=[
            pl.BlockSpec((1, NUM_COMBO, D_MODEL),
                         lambda i: ((i * TC_BLK) // BATCH, 0, 0)),
            pl.BlockSpec((1, 1, TC_BLK), lambda i: (i, 0, 0)),
        ],
        out_specs=pl.BlockSpec((TC_BLK, D_MODEL), lambda i: (i, 0)),
        out_shape=jax.ShapeDtypeStruct((TC_ROWS, D_MODEL), jnp.float32),
    )(ta, ci_blocks)


def _sc_gather(table, idx):
    mesh = plsc.VectorSubcoreMesh(core_axis_name="core",
                                  subcore_axis_name="subcore")

    @functools.partial(
        pl.kernel,
        out_type=jax.ShapeDtypeStruct((SC_ROWS, D_MODEL), jnp.float32),
        mesh=mesh,
        scratch_types=[
            pltpu.VMEM_SHARED((NUM_COMBO * MEMORY_LEN, D_MODEL), jnp.float32),
        ],
    )
    def k(t_hbm, i_hbm, o_hbm, t_spmem):
        @pl.when(lax.axis_index("subcore") == 0)
        def _():
            pltpu.sync_copy(t_hbm, t_spmem)

        plsc.subcore_barrier()

        def body(i_vmem, o_vmem):
            pltpu.sync_copy(t_spmem.at[i_vmem.at[0]],
                            o_vmem.at[pl.ds(0, WINDOW)])
            pltpu.sync_copy(t_spmem.at[i_vmem.at[1]],
                            o_vmem.at[pl.ds(WINDOW, WINDOW)])

        pltpu.emit_pipeline(
            body,
            grid=(SC_ROWS // (2 * WINDOW),),
            in_specs=[pl.BlockSpec((2, WINDOW), index_map=lambda i: (i, 0))],
            out_specs=[pl.BlockSpec((2 * WINDOW, D_MODEL),
                                    index_map=lambda i: (i, 0))],
            core_axis_name=("core", "subcore"),
            dimension_semantics=(pltpu.PARALLEL,),
        )(i_hbm, o_hbm)

    return k(table, idx)


def kernel(x_pitch, x_label, pos_table, pitch_table, label_table,
           ln_gamma, ln_beta):
    xp_t = x_pitch.T
    xl_t = x_label.T
    t, ta, combo, ci = _prep(pos_table, pitch_table, label_table,
                             ln_gamma.reshape(1, D_MODEL),
                             ln_beta.reshape(1, D_MODEL), xp_t, xl_t)
    ci_blocks = ci.reshape(ROWS // TC_BLK, 1, TC_BLK)[:TC_ROWS // TC_BLK]
    tc_out = _tc_gather(ta, ci_blocks)
    sc_idx = combo.reshape(GRID, WINDOW)[TC_ROWS // WINDOW:]
    sc_out = _sc_gather(t.reshape(NUM_COMBO * MEMORY_LEN, D_MODEL), sc_idx)
    out_flat = jnp.concatenate([tc_out, sc_out], axis=0)
    return out_flat.reshape(MEMORY_LEN, BATCH, D_MODEL)


# speedup vs baseline: 1.5151x; 1.5151x over previous
import functools

import numpy as np
import jax
import jax.numpy as jnp
from jax import lax
from jax.experimental import pallas as pl
from jax.experimental.pallas import tpu as pltpu
from jax.experimental.pallas import tpu_sc as plsc

D_MODEL = 128
MEMORY_LEN = 200
BATCH = 4096
NUM_COMBO = 25
SCALE = float(np.sqrt(D_MODEL))
EPS = 1e-5
ROWS = MEMORY_LEN * BATCH
WINDOW = 128
GRID = ROWS // WINDOW

TC_ROWS = 409600
SC_ROWS = ROWS - TC_ROWS
TC_BLK = 512
SC_STEP = 2 * WINDOW
SC_OUT_BLOCK_OFF = TC_ROWS // SC_STEP


def _prep_body(pos_ref, pt_ref, lt_ref, g_ref, b_ref, xp_ref, xl_ref,
               t_ref, hi_ref, lo_ref, combo_ref, ci_ref):
    gamma = g_ref[...]
    beta = b_ref[...]
    pos2 = 2.0 * pos_ref[...]
    for c in range(NUM_COMBO):
        p, lab = c // 5, c % 5
        row = SCALE * (pt_ref[p:p + 1, :] + lt_ref[lab:lab + 1, :])
        x = pos2 + row
        mean = jnp.mean(x, axis=-1, keepdims=True)
        var = jnp.mean((x - mean) ** 2, axis=-1, keepdims=True)
        y = (x - mean) * lax.rsqrt(var + EPS) * gamma + beta
        t_ref[c] = y
        hi = y.astype(jnp.bfloat16)
        hi_ref[:, c, :] = hi
        lo_ref[:, c, :] = (y - hi.astype(jnp.float32)).astype(jnp.bfloat16)
    ci = 5 * xp_ref[...] + xl_ref[...]
    ci_ref[...] = ci
    iota_l = lax.broadcasted_iota(jnp.int32, (MEMORY_LEN, BATCH), 0)
    combo_ref[...] = ci * MEMORY_LEN + iota_l


def _prep(pos_table, pitch_table, label_table, ln_gamma, ln_beta, xp_t, xl_t):
    return pl.pallas_call(
        _prep_body,
        out_shape=[
            jax.ShapeDtypeStruct((NUM_COMBO, MEMORY_LEN, D_MODEL), jnp.float32),
            jax.ShapeDtypeStruct((MEMORY_LEN, NUM_COMBO, D_MODEL),
                                 jnp.bfloat16),
            jax.ShapeDtypeStruct((MEMORY_LEN, NUM_COMBO, D_MODEL),
                                 jnp.bfloat16),
            jax.ShapeDtypeStruct((MEMORY_LEN, BATCH), jnp.int32),
            jax.ShapeDtypeStruct((MEMORY_LEN, BATCH), jnp.int32),
        ],
    )(pos_table, pitch_table, label_table, ln_gamma, ln_beta, xp_t, xl_t)


def _tc_gather_body(sc_ref, hi_ref, lo_ref, ci_ref, o_ref):
    del sc_ref
    ci = ci_ref[0, 0]
    onehot = (ci[:, None] == lax.broadcasted_iota(
        jnp.int32, (TC_BLK, NUM_COMBO), 1)).astype(jnp.bfloat16)
    o_ref[...] = (jnp.dot(onehot, hi_ref[0],
                          preferred_element_type=jnp.float32) +
                  jnp.dot(onehot, lo_ref[0],
                          preferred_element_type=jnp.float32))


def _tc_gather(sc_buf, hi, lo, ci_blocks):
    n_blk = TC_ROWS // TC_BLK
    return pl.pallas_call(
        _tc_gather_body,
        grid=(n_blk,),
        in_specs=[
            pl.BlockSpec(memory_space=pl.ANY),
            pl.BlockSpec((1, NUM_COMBO, D_MODEL),
                         lambda i: ((i * TC_BLK) // BATCH, 0, 0)),
            pl.BlockSpec((1, NUM_COMBO, D_MODEL),
                         lambda i: ((i * TC_BLK) // BATCH, 0, 0)),
            pl.BlockSpec((1, 1, TC_BLK), lambda i: (i, 0, 0)),
        ],
        out_specs=pl.BlockSpec((TC_BLK, D_MODEL), lambda i: (i, 0)),
        out_shape=jax.ShapeDtypeStruct((ROWS, D_MODEL), jnp.float32),
        input_output_aliases={0: 0},
    )(sc_buf, hi, lo, ci_blocks)


def _sc_gather(table, idx):
    mesh = plsc.VectorSubcoreMesh(core_axis_name="core",
                                  subcore_axis_name="subcore")

    @functools.partial(
        pl.kernel,
        out_type=jax.ShapeDtypeStruct((ROWS, D_MODEL), jnp.float32),
        mesh=mesh,
        scratch_types=[
            pltpu.VMEM_SHARED((NUM_COMBO * MEMORY_LEN, D_MODEL), jnp.float32),
        ],
    )
    def k(t_hbm, i_hbm, o_hbm, t_spmem):
        @pl.when(lax.axis_index("subcore") == 0)
        def _():
            pltpu.sync_copy(t_hbm, t_spmem)

        plsc.subcore_barrier()

        def body(i_vmem, o_vmem):
            pltpu.sync_copy(t_spmem.at[i_vmem.at[0]],
                            o_vmem.at[pl.ds(0, WINDOW)])
            pltpu.sync_copy(t_spmem.at[i_vmem.at[1]],
                            o_vmem.at[pl.ds(WINDOW, WINDOW)])

        pltpu.emit_pipeline(
            body,
            grid=(SC_ROWS // SC_STEP,),
            in_specs=[pl.BlockSpec((2, WINDOW), index_map=lambda i: (i, 0))],
            out_specs=[pl.BlockSpec((SC_STEP, D_MODEL),
                                    index_map=lambda i: (SC_OUT_BLOCK_OFF + i,
                                                         0))],
            core_axis_name=("core", "subcore"),
            dimension_semantics=(pltpu.PARALLEL,),
        )(i_hbm, o_hbm)

    return k(table, idx)


def kernel(x_pitch, x_label, pos_table, pitch_table, label_table,
           ln_gamma, ln_beta):
    xp_t = x_pitch.T
    xl_t = x_label.T
    t, hi, lo, combo, ci = _prep(pos_table, pitch_table, label_table,
                                 ln_gamma.reshape(1, D_MODEL),
                                 ln_beta.reshape(1, D_MODEL), xp_t, xl_t)
    sc_idx = combo.reshape(GRID, WINDOW)[TC_ROWS // WINDOW:]
    sc_buf = _sc_gather(t.reshape(NUM_COMBO * MEMORY_LEN, D_MODEL), sc_idx)
    ci_blocks = ci.reshape(ROWS // TC_BLK, 1, TC_BLK)[:TC_ROWS // TC_BLK]
    out_flat = _tc_gather(sc_buf, hi, lo, ci_blocks)
    return out_flat.reshape(MEMORY_LEN, BATCH, D_MODEL)


# speedup vs baseline: 3.0411x; 2.0072x over previous
import functools

import numpy as np
import jax
import jax.numpy as jnp
from jax import lax
from jax.experimental import pallas as pl
from jax.experimental.pallas import tpu as pltpu
from jax.experimental.pallas import tpu_sc as plsc

D_MODEL = 128
MEMORY_LEN = 200
BATCH = 4096
NUM_COMBO = 25
SCALE = float(np.sqrt(D_MODEL))
EPS = 1e-5
ROWS = MEMORY_LEN * BATCH
WINDOW = 128
GRID = ROWS // WINDOW

TC_ROWS = 409600
SC_ROWS = ROWS - TC_ROWS
TC_BLK = 2048
SC_STEP = 2 * WINDOW
SC_OUT_BLOCK_OFF = TC_ROWS // SC_STEP


def _prep_body(pos_ref, pt_ref, lt_ref, g_ref, b_ref, xp_ref, xl_ref,
               t_ref, hi_ref, lo_ref, combo_ref, ci_ref):
    gamma = g_ref[...]
    beta = b_ref[...]
    pos2 = 2.0 * pos_ref[...]
    for c in range(NUM_COMBO):
        p, lab = c // 5, c % 5
        row = SCALE * (pt_ref[p:p + 1, :] + lt_ref[lab:lab + 1, :])
        x = pos2 + row
        mean = jnp.mean(x, axis=-1, keepdims=True)
        var = jnp.mean((x - mean) ** 2, axis=-1, keepdims=True)
        y = (x - mean) * lax.rsqrt(var + EPS) * gamma + beta
        t_ref[c] = y
        hi = y.astype(jnp.bfloat16)
        hi_ref[:, c, :] = hi
        lo_ref[:, c, :] = (y - hi.astype(jnp.float32)).astype(jnp.bfloat16)
    ci = 5 * xp_ref[...] + xl_ref[...]
    ci_ref[...] = ci
    iota_l = lax.broadcasted_iota(jnp.int32, (MEMORY_LEN, BATCH), 0)
    combo_ref[...] = ci * MEMORY_LEN + iota_l


def _prep(pos_table, pitch_table, label_table, ln_gamma, ln_beta, xp_t, xl_t):
    return pl.pallas_call(
        _prep_body,
        out_shape=[
            jax.ShapeDtypeStruct((NUM_COMBO, MEMORY_LEN, D_MODEL), jnp.float32),
            jax.ShapeDtypeStruct((MEMORY_LEN, NUM_COMBO, D_MODEL),
                                 jnp.bfloat16),
            jax.ShapeDtypeStruct((MEMORY_LEN, NUM_COMBO, D_MODEL),
                                 jnp.bfloat16),
            jax.ShapeDtypeStruct((MEMORY_LEN, BATCH), jnp.int32),
            jax.ShapeDtypeStruct((MEMORY_LEN, BATCH), jnp.int32),
        ],
    )(pos_table, pitch_table, label_table, ln_gamma, ln_beta, xp_t, xl_t)


def _tc_gather_body(sc_ref, hi_ref, lo_ref, ci_ref, o_ref):
    del sc_ref
    ci = ci_ref[0, 0]
    onehot = (ci[:, None] == lax.broadcasted_iota(
        jnp.int32, (TC_BLK, NUM_COMBO), 1)).astype(jnp.bfloat16)
    o_ref[...] = (jnp.dot(onehot, hi_ref[0],
                          preferred_element_type=jnp.float32) +
                  jnp.dot(onehot, lo_ref[0],
                          preferred_element_type=jnp.float32))


def _tc_gather(sc_buf, hi, lo, ci_blocks):
    n_blk = TC_ROWS // TC_BLK
    return pl.pallas_call(
        _tc_gather_body,
        grid=(n_blk,),
        in_specs=[
            pl.BlockSpec(memory_space=pl.ANY),
            pl.BlockSpec((1, NUM_COMBO, D_MODEL),
                         lambda i: ((i * TC_BLK) // BATCH, 0, 0)),
            pl.BlockSpec((1, NUM_COMBO, D_MODEL),
                         lambda i: ((i * TC_BLK) // BATCH, 0, 0)),
            pl.BlockSpec((1, 1, TC_BLK), lambda i: (i, 0, 0)),
        ],
        out_specs=pl.BlockSpec((TC_BLK, D_MODEL), lambda i: (i, 0)),
        out_shape=jax.ShapeDtypeStruct((ROWS, D_MODEL), jnp.float32),
        input_output_aliases={0: 0},
    )(sc_buf, hi, lo, ci_blocks)


def _sc_gather(table, idx):
    mesh = plsc.VectorSubcoreMesh(core_axis_name="core",
                                  subcore_axis_name="subcore")

    @functools.partial(
        pl.kernel,
        out_type=jax.ShapeDtypeStruct((ROWS, D_MODEL), jnp.float32),
        mesh=mesh,
        scratch_types=[
            pltpu.VMEM_SHARED((NUM_COMBO * MEMORY_LEN, D_MODEL), jnp.float32),
        ],
    )
    def k(t_hbm, i_hbm, o_hbm, t_spmem):
        @pl.when(lax.axis_index("subcore") == 0)
        def _():
            pltpu.sync_copy(t_hbm, t_spmem)

        plsc.subcore_barrier()

        def body(i_vmem, o_vmem):
            pltpu.sync_copy(t_spmem.at[i_vmem.at[0]],
                            o_vmem.at[pl.ds(0, WINDOW)])
            pltpu.sync_copy(t_spmem.at[i_vmem.at[1]],
                            o_vmem.at[pl.ds(WINDOW, WINDOW)])

        pltpu.emit_pipeline(
            body,
            grid=(SC_ROWS // SC_STEP,),
            in_specs=[pl.BlockSpec((2, WINDOW), index_map=lambda i: (i, 0))],
            out_specs=[pl.BlockSpec((SC_STEP, D_MODEL),
                                    index_map=lambda i: (SC_OUT_BLOCK_OFF + i,
                                                         0))],
            core_axis_name=("core", "subcore"),
            dimension_semantics=(pltpu.PARALLEL,),
        )(i_hbm, o_hbm)

    return k(table, idx)


def kernel(x_pitch, x_label, pos_table, pitch_table, label_table,
           ln_gamma, ln_beta):
    xp_t = x_pitch.T
    xl_t = x_label.T
    t, hi, lo, combo, ci = _prep(pos_table, pitch_table, label_table,
                                 ln_gamma.reshape(1, D_MODEL),
                                 ln_beta.reshape(1, D_MODEL), xp_t, xl_t)
    sc_idx = combo.reshape(GRID, WINDOW)[TC_ROWS // WINDOW:]
    sc_buf = _sc_gather(t.reshape(NUM_COMBO * MEMORY_LEN, D_MODEL), sc_idx)
    ci_blocks = ci.reshape(ROWS // TC_BLK, 1, TC_BLK)[:TC_ROWS // TC_BLK]
    out_flat = _tc_gather(sc_buf, hi, lo, ci_blocks)
    return out_flat.reshape(MEMORY_LEN, BATCH, D_MODEL)


# speedup vs baseline: 4.4363x; 1.4588x over previous
import functools

import numpy as np
import jax
import jax.numpy as jnp
from jax import lax
from jax.experimental import pallas as pl
from jax.experimental.pallas import tpu as pltpu
from jax.experimental.pallas import tpu_sc as plsc

D_MODEL = 128
MEMORY_LEN = 200
BATCH = 4096
NUM_COMBO = 25
SCALE = float(np.sqrt(D_MODEL))
EPS = 1e-5
ROWS = MEMORY_LEN * BATCH
WINDOW = 128
GRID = ROWS // WINDOW


def _prep_body(pos_ref, pt_ref, lt_ref, g_ref, b_ref, xp_ref, xl_ref,
               t_ref, combo_ref):
    gamma = g_ref[...]
    beta = b_ref[...]
    pos2 = 2.0 * pos_ref[...]
    for c in range(NUM_COMBO):
        p, lab = c // 5, c % 5
        row = SCALE * (pt_ref[p:p + 1, :] + lt_ref[lab:lab + 1, :])
        x = pos2 + row
        mean = jnp.mean(x, axis=-1, keepdims=True)
        var = jnp.mean((x - mean) ** 2, axis=-1, keepdims=True)
        t_ref[c] = (x - mean) * lax.rsqrt(var + EPS) * gamma + beta
    iota_l = lax.broadcasted_iota(jnp.int32, (MEMORY_LEN, BATCH), 0)
    combo_ref[...] = (5 * xp_ref[...] + xl_ref[...]) * MEMORY_LEN + iota_l


def _prep(pos_table, pitch_table, label_table, ln_gamma, ln_beta, xp_t, xl_t):
    return pl.pallas_call(
        _prep_body,
        out_shape=[
            jax.ShapeDtypeStruct((NUM_COMBO, MEMORY_LEN, D_MODEL), jnp.float32),
            jax.ShapeDtypeStruct((MEMORY_LEN, BATCH), jnp.int32),
        ],
    )(pos_table, pitch_table, label_table, ln_gamma, ln_beta, xp_t, xl_t)


def _gather(table, idx):
    mesh = plsc.VectorSubcoreMesh(core_axis_name="core",
                                  subcore_axis_name="subcore")

    @functools.partial(
        pl.kernel,
        out_type=jax.ShapeDtypeStruct((ROWS, D_MODEL), jnp.float32),
        mesh=mesh,
        scratch_types=[
            pltpu.VMEM_SHARED((NUM_COMBO * MEMORY_LEN, D_MODEL), jnp.float32),
        ],
    )
    def k(t_hbm, i_hbm, o_hbm, t_spmem):
        sid = lax.axis_index("subcore")

        @pl.when(sid < 15)
        def _():
            pltpu.sync_copy(t_hbm.at[pl.ds(sid * 312, 312)],
                            t_spmem.at[pl.ds(sid * 312, 312)])

        @pl.when(sid == 15)
        def _():
            pltpu.sync_copy(t_hbm.at[pl.ds(4680, 320)],
                            t_spmem.at[pl.ds(4680, 320)])

        plsc.subcore_barrier()

        def body(i_vmem, o_vmem):
            pltpu.sync_copy(t_spmem.at[i_vmem.at[0]],
                            o_vmem.at[pl.ds(0, WINDOW)])
            pltpu.sync_copy(t_spmem.at[i_vmem.at[1]],
                            o_vmem.at[pl.ds(WINDOW, WINDOW)])

        pltpu.emit_pipeline(
            body,
            grid=(GRID // 2,),
            in_specs=[pl.BlockSpec((2, WINDOW), index_map=lambda i: (i, 0))],
            out_specs=[pl.BlockSpec((2 * WINDOW, D_MODEL),
                                    index_map=lambda i: (i, 0))],
            core_axis_name=("core", "subcore"),
            dimension_semantics=(pltpu.PARALLEL,),
        )(i_hbm, o_hbm)

    return k(table, idx)


def kernel(x_pitch, x_label, pos_table, pitch_table, label_table,
           ln_gamma, ln_beta):
    xp_t = x_pitch.T
    xl_t = x_label.T
    t, combo = _prep(pos_table, pitch_table, label_table,
                     ln_gamma.reshape(1, D_MODEL), ln_beta.reshape(1, D_MODEL),
                     xp_t, xl_t)
    out_flat = _gather(t.reshape(NUM_COMBO * MEMORY_LEN, D_MODEL),
                       combo.reshape(GRID, WINDOW))
    return out_flat.reshape(MEMORY_LEN, BATCH, D_MODEL)
